# baseline (speedup 1.0000x reference)
import functools

import jax
import jax.numpy as jnp
from jax import lax
from jax.experimental import pallas as pl
from jax.experimental.pallas import tpu as pltpu
from jax.experimental.pallas import tpu_sc as plsc

N = 10000
E = 160000
F_IN = 256
DIM = 512
G = 64

BM = 2000

NC = 2
NS = 16
LANES = 16

WROWS = 160
NWIN = 63
NROUND = 2
ACC_ROWS = 161
NOUT = NWIN * WROWS
CHUNK = 4000
NCHK = E // CHUNK
GRP = CHUNK // LANES
GB = 32
PIPE = 2
SLOTW = CHUNK + 3 * LANES
DUMMY = WROWS << 16


def _sc_part_body(src_hbm, dst_hbm, packed_hbm,
                  srcb, dstb, cpk0a, cpk0b, cpk1a, cpk1b, seme, semw):
    c = lax.axis_index("c")
    s = lax.axis_index("s")
    wid = c * NS + s
    base0 = wid * WROWS
    base1 = (NC * NS + wid) * WROWS
    lanesv = lax.iota(jnp.int32, LANES)
    hdr0 = jnp.full((LANES,), LANES, jnp.int32)

    def echunk(k, par):
        off = par * CHUNK
        return (pltpu.make_async_copy(
                    dst_hbm.at[pl.ds(k * CHUNK, CHUNK)],
                    dstb.at[pl.ds(off, CHUNK)], seme),
                pltpu.make_async_copy(
                    src_hbm.at[pl.ds(k * CHUNK, CHUNK)],
                    srcb.at[pl.ds(off, CHUNK)], seme))

    for cp in echunk(0, 0):
        cp.start()

    def pair_body(kk, carry):
        for par, cpk0, cpk1 in ((0, cpk0a, cpk1a), (1, cpk0b, cpk1b)):
            k = 2 * kk + par
            eoff = par * CHUNK
            for cp in echunk(k, par):
                cp.wait()

            @pl.when(k + 1 < NCHK)
            def _():
                for cp in echunk(k + 1, 1 - par):
                    cp.start()

            @pl.when(kk > 0)
            def _():
                pltpu.make_async_copy(
                    packed_hbm.at[pl.ds(0, SLOTW)], cpk0, semw).wait()
                pltpu.make_async_copy(
                    packed_hbm.at[pl.ds(0, SLOTW)], cpk1, semw).wait()

            def fbody(i, nvs):
                nv0, nv1 = nvs
                dv = dstb[pl.ds(eoff + i * LANES, LANES)]
                sv = srcb[pl.ds(eoff + i * LANES, LANES)]
                lo0 = dv - base0
                m0 = plsc.bitcast(lo0, jnp.uint32) < WROWS
                pos0 = nv0 + plsc.cumsum(jnp.where(m0, 1, 0)) - 1
                p0 = lax.shift_left(lo0, 16) | sv
                plsc.store_scatter(cpk0, [pos0], p0, mask=m0)
                lo1 = dv - base1
                m1 = plsc.bitcast(lo1, jnp.uint32) < WROWS
                pos1 = nv1 + plsc.cumsum(jnp.where(m1, 1, 0)) - 1
                p1 = lax.shift_left(lo1, 16) | sv
                plsc.store_scatter(cpk1, [pos1], p1, mask=m1)
                return (nv0 + plsc.all_reduce_population_count(m0),
                        nv1 + plsc.all_reduce_population_count(m1))

            nv0, nv1 = lax.fori_loop(0, GRP, fbody, (hdr0, hdr0))

            for cpk, nv, w in ((cpk0, nv0, wid), (cpk1, nv1, NC * NS + wid)):
                n = nv[0]
                g0 = n // LANES
                keep = lanesv < (n - g0 * LANES)
                vd = cpk[pl.ds(g0 * LANES, LANES)]
                cpk[pl.ds(g0 * LANES, LANES)] = jnp.where(keep, vd, DUMMY)
                cpk[pl.ds((g0 + 1) * LANES, LANES)] = jnp.full(
                    (LANES,), DUMMY, jnp.int32)
                cpk[pl.ds(0, LANES)] = (nv - LANES + GB - 1) // GB
                pltpu.make_async_copy(
                    cpk, packed_hbm.at[pl.ds((w * NCHK + k) * SLOTW, SLOTW)],
                    semw).start()
        return carry

    lax.fori_loop(0, NCHK // 2, pair_body, 0)

    for cpk in (cpk0a, cpk1a, cpk0b, cpk1b):
        pltpu.make_async_copy(
            packed_hbm.at[pl.ds(0, SLOTW)], cpk, semw).wait()


def _partition(src, dst):
    mesh = plsc.VectorSubcoreMesh(core_axis_name="c", subcore_axis_name="s",
                                  num_cores=NC, num_subcores=NS)
    k = pl.kernel(
        _sc_part_body,
        out_type=jax.ShapeDtypeStruct((2 * NC * NS * NCHK * SLOTW,), jnp.int32),
        mesh=mesh,
        compiler_params=pltpu.CompilerParams(needs_layout_passes=False),
        scratch_types=[
            pltpu.VMEM((2 * CHUNK,), jnp.int32),
            pltpu.VMEM((2 * CHUNK,), jnp.int32),
            pltpu.VMEM((SLOTW,), jnp.int32),
            pltpu.VMEM((SLOTW,), jnp.int32),
            pltpu.VMEM((SLOTW,), jnp.int32),
            pltpu.VMEM((SLOTW,), jnp.int32),
            pltpu.SemaphoreType.DMA,
            pltpu.SemaphoreType.DMA,
        ],
    )
    return k(src, dst)


def _sc_agg_body(h_hbm, packed_hbm, out_hbm,
                 slotb, rows, idxring, acc, semg, sems):
    c = lax.axis_index("c")
    s = lax.axis_index("s")
    wid = c * NS + s
    d = rows.shape[1]
    zero = jnp.zeros((LANES,), jnp.float32)

    for r in range(NROUND):
        w = r * (NC * NS) + wid

        @pl.when(w < NWIN)
        def _():
            base = w * WROWS

            def zbody(i, carry):
                for j in range(d // LANES):
                    acc[i, pl.ds(j * LANES, LANES)] = zero
                return carry

            lax.fori_loop(0, ACC_ROWS, zbody, 0)

            def stage(k):
                return pltpu.make_async_copy(
                    packed_hbm.at[pl.ds((w * NCHK + k) * SLOTW, SLOTW)],
                    slotb.at[pl.ds((k % 2) * SLOTW, SLOTW)], sems)

            stage(0).start()

            def slot_body(k, carry):
                soff = (k % 2) * SLOTW
                stage(k).wait()

                @pl.when(k + 1 < NCHK)
                def _():
                    stage(k + 1).start()

                nb = slotb[pl.ds(soff, LANES)][0]

                def gcopy(b):
                    return pltpu.make_async_copy(
                        h_hbm.at[idxring.at[pl.ds((b % PIPE) * GB, GB)]],
                        rows.at[pl.ds((b % PIPE) * GB, GB)], semg)

                def prep(b):
                    for g in range(GB // LANES):
                        pv = slotb[pl.ds(soff + LANES + b * GB + g * LANES,
                                         LANES)]
                        idxring[pl.ds((b % PIPE) * GB + g * LANES,
                                      LANES)] = pv & 0xFFFF
                    gcopy(b).start()

                def prebody(b, carry):
                    prep(b)
                    return carry

                lax.fori_loop(0, jnp.minimum(nb, PIPE), prebody, 0)

                def gbody(b, carry):
                    gcopy(b).wait()
                    off = (b % PIPE) * GB
                    for g in range(GB // LANES):
                        pv = slotb[pl.ds(soff + LANES + b * GB + g * LANES,
                                         LANES)]
                        ldstv = lax.shift_right_logical(pv, 16)
                        for kk in range(LANES):
                            row = ldstv[kk]
                            for j in range(d // LANES):
                                plsc.addupdate(
                                    acc.at[row, pl.ds(j * LANES, LANES)],
                                    rows[off + g * LANES + kk,
                                         pl.ds(j * LANES, LANES)])

                    @pl.when(b + PIPE < nb)
                    def _():
                        prep(b + PIPE)

                    return carry

                lax.fori_loop(0, nb, gbody, 0)
                return carry

            lax.fori_loop(0, NCHK, slot_body, 0)

            pltpu.sync_copy(acc.at[pl.ds(0, WROWS)],
                            out_hbm.at[pl.ds(base, WROWS)])


def _aggregate(h, packed):
    d = h.shape[1]
    mesh = plsc.VectorSubcoreMesh(core_axis_name="c", subcore_axis_name="s",
                                  num_cores=NC, num_subcores=NS)
    k = pl.kernel(
        _sc_agg_body,
        out_type=jax.ShapeDtypeStruct((NOUT, d), jnp.float32),
        mesh=mesh,
        compiler_params=pltpu.CompilerParams(needs_layout_passes=False),
        scratch_types=[
            pltpu.VMEM((2 * SLOTW,), jnp.int32),
            pltpu.VMEM((PIPE * GB, d), jnp.float32),
            pltpu.VMEM((PIPE * GB,), jnp.int32),
            pltpu.VMEM((ACC_ROWS, d), jnp.float32),
            pltpu.SemaphoreType.DMA,
            pltpu.SemaphoreType.DMA,
        ],
    )
    return k(h, packed)[:N]


def _mlp_body(h_ref, agg_ref, w1_ref, b1_ref, w2_ref, b2_ref, o_ref):
    z = h_ref[...] + agg_ref[...]
    t = jnp.dot(z, w1_ref[...], preferred_element_type=jnp.float32)
    t = jnp.maximum(t + b1_ref[...], 0.0)
    o = jnp.dot(t, w2_ref[...], preferred_element_type=jnp.float32)
    o_ref[...] = jnp.maximum(o + b2_ref[...], 0.0)


def _mlp(h, agg, w1, b1, w2, b2):
    din = h.shape[1]
    grid = N // BM
    return pl.pallas_call(
        _mlp_body,
        grid=(grid,),
        in_specs=[
            pl.BlockSpec((BM, din), lambda i: (i, 0)),
            pl.BlockSpec((BM, din), lambda i: (i, 0)),
            pl.BlockSpec((din, DIM), lambda i: (0, 0)),
            pl.BlockSpec((1, DIM), lambda i: (0, 0)),
            pl.BlockSpec((DIM, DIM), lambda i: (0, 0)),
            pl.BlockSpec((1, DIM), lambda i: (0, 0)),
        ],
        out_specs=pl.BlockSpec((BM, DIM), lambda i: (i, 0)),
        out_shape=jax.ShapeDtypeStruct((N, DIM), jnp.float32),
    )(h, agg, w1, b1.reshape(1, DIM), w2, b2.reshape(1, DIM))


def _mlp_pool_body(h_ref, agg_ref, w1_ref, b1_ref, w2_ref, b2_ref,
                   batch_ref, o_ref):
    z = h_ref[...] + agg_ref[...]
    t = jnp.dot(z, w1_ref[...], preferred_element_type=jnp.float32)
    t = jnp.maximum(t + b1_ref[...], 0.0)
    o = jnp.dot(t, w2_ref[...], preferred_element_type=jnp.float32)
    o = jnp.maximum(o + b2_ref[...], 0.0)
    bids = batch_ref[0]
    onehot = (jax.lax.broadcasted_iota(jnp.int32, (G, BM), 0) == bids).astype(
        jnp.float32)
    contrib = jnp.dot(onehot, o, preferred_element_type=jnp.float32)

    @pl.when(pl.program_id(0) == 0)
    def _():
        o_ref[...] = contrib

    @pl.when(pl.program_id(0) != 0)
    def _():
        o_ref[...] += contrib


def _mlp_pool(h, agg, w1, b1, w2, b2, batch3):
    din = h.shape[1]
    grid = N // BM
    return pl.pallas_call(
        _mlp_pool_body,
        grid=(grid,),
        in_specs=[
            pl.BlockSpec((BM, din), lambda i: (i, 0)),
            pl.BlockSpec((BM, din), lambda i: (i, 0)),
            pl.BlockSpec((din, DIM), lambda i: (0, 0)),
            pl.BlockSpec((1, DIM), lambda i: (0, 0)),
            pl.BlockSpec((DIM, DIM), lambda i: (0, 0)),
            pl.BlockSpec((1, DIM), lambda i: (0, 0)),
            pl.BlockSpec((1, 1, BM), lambda i: (i, 0, 0)),
        ],
        out_specs=pl.BlockSpec((G, DIM), lambda i: (0, 0)),
        out_shape=jax.ShapeDtypeStruct((G, DIM), jnp.float32),
    )(h, agg, w1, b1.reshape(1, DIM), w2, b2.reshape(1, DIM), batch3)


def kernel(x, edge_index, batch,
           W1_0, b1_0, W2_0, b2_0,
           W1_1, b1_1, W2_1, b2_1,
           W1_2, b1_2, W2_2, b2_2):
    src = edge_index[0].astype(jnp.int32)
    dst = edge_index[1].astype(jnp.int32)
    batch3 = batch.astype(jnp.int32).reshape(N // BM, 1, BM)

    packed = _partition(src, dst)
    h = x
    agg = _aggregate(h, packed)
    h = _mlp(h, agg, W1_0, b1_0, W2_0, b2_0)
    agg = _aggregate(h, packed)
    h = _mlp(h, agg, W1_1, b1_1, W2_1, b2_1)
    agg = _aggregate(h, packed)
    return _mlp_pool(h, agg, W1_2, b1_2, W2_2, b2_2, batch3)


# speedup vs baseline: 1.7619x; 1.7619x over previous
import functools

import jax
import jax.numpy as jnp
from jax import lax
from jax.experimental import pallas as pl
from jax.experimental.pallas import tpu as pltpu
from jax.experimental.pallas import tpu_sc as plsc

N = 10000
E = 160000
F_IN = 256
DIM = 512
G = 64

BM = 2000

NC = 2
NS = 16
LANES = 16

WROWS = 160
NWIN = 63
NROUND = 2
ACC_ROWS = 161
NOUT = NWIN * WROWS
CHUNK = 4000
NCHK = E // CHUNK
GRP = CHUNK // LANES
GB = 16
PIPE = 4
SLOTW = CHUNK + 3 * LANES
DUMMY = WROWS << 16


def _sc_part_body(src_hbm, dst_hbm, packed_hbm,
                  srcb, dstb, cpk0a, cpk0b, cpk1a, cpk1b, seme, semw):
    c = lax.axis_index("c")
    s = lax.axis_index("s")
    wid = c * NS + s
    base0 = wid * WROWS
    base1 = (NC * NS + wid) * WROWS
    lanesv = lax.iota(jnp.int32, LANES)
    hdr0 = jnp.full((LANES,), LANES, jnp.int32)

    def echunk(k, par):
        off = par * CHUNK
        return (pltpu.make_async_copy(
                    dst_hbm.at[pl.ds(k * CHUNK, CHUNK)],
                    dstb.at[pl.ds(off, CHUNK)], seme),
                pltpu.make_async_copy(
                    src_hbm.at[pl.ds(k * CHUNK, CHUNK)],
                    srcb.at[pl.ds(off, CHUNK)], seme))

    for cp in echunk(0, 0):
        cp.start()

    def pair_body(kk, carry):
        for par, cpk0, cpk1 in ((0, cpk0a, cpk1a), (1, cpk0b, cpk1b)):
            k = 2 * kk + par
            eoff = par * CHUNK
            for cp in echunk(k, par):
                cp.wait()

            @pl.when(k + 1 < NCHK)
            def _():
                for cp in echunk(k + 1, 1 - par):
                    cp.start()

            @pl.when(kk > 0)
            def _():
                pltpu.make_async_copy(
                    packed_hbm.at[pl.ds(0, SLOTW)], cpk0, semw).wait()
                pltpu.make_async_copy(
                    packed_hbm.at[pl.ds(0, SLOTW)], cpk1, semw).wait()

            def fbody(i, nvs):
                nv0, nv1 = nvs
                dv = dstb[pl.ds(eoff + i * LANES, LANES)]
                sv = srcb[pl.ds(eoff + i * LANES, LANES)]
                lo0 = dv - base0
                m0 = plsc.bitcast(lo0, jnp.uint32) < WROWS
                pos0 = nv0 + plsc.cumsum(jnp.where(m0, 1, 0)) - 1
                p0 = lax.shift_left(lo0, 16) | sv
                plsc.store_scatter(cpk0, [pos0], p0, mask=m0)
                lo1 = dv - base1
                m1 = plsc.bitcast(lo1, jnp.uint32) < WROWS
                pos1 = nv1 + plsc.cumsum(jnp.where(m1, 1, 0)) - 1
                p1 = lax.shift_left(lo1, 16) | sv
                plsc.store_scatter(cpk1, [pos1], p1, mask=m1)
                return (nv0 + plsc.all_reduce_population_count(m0),
                        nv1 + plsc.all_reduce_population_count(m1))

            nv0, nv1 = lax.fori_loop(0, GRP, fbody, (hdr0, hdr0))

            for cpk, nv, w in ((cpk0, nv0, wid), (cpk1, nv1, NC * NS + wid)):
                n = nv[0]
                g0 = n // LANES
                keep = lanesv < (n - g0 * LANES)
                vd = cpk[pl.ds(g0 * LANES, LANES)]
                cpk[pl.ds(g0 * LANES, LANES)] = jnp.where(keep, vd, DUMMY)
                cpk[pl.ds((g0 + 1) * LANES, LANES)] = jnp.full(
                    (LANES,), DUMMY, jnp.int32)
                cpk[pl.ds(0, LANES)] = (nv - LANES + GB - 1) // GB
                pltpu.make_async_copy(
                    cpk, packed_hbm.at[pl.ds((w * NCHK + k) * SLOTW, SLOTW)],
                    semw).start()
        return carry

    lax.fori_loop(0, NCHK // 2, pair_body, 0)

    for cpk in (cpk0a, cpk1a, cpk0b, cpk1b):
        pltpu.make_async_copy(
            packed_hbm.at[pl.ds(0, SLOTW)], cpk, semw).wait()


def _partition(src, dst):
    mesh = plsc.VectorSubcoreMesh(core_axis_name="c", subcore_axis_name="s",
                                  num_cores=NC, num_subcores=NS)
    k = pl.kernel(
        _sc_part_body,
        out_type=jax.ShapeDtypeStruct((2 * NC * NS * NCHK * SLOTW,), jnp.int32),
        mesh=mesh,
        compiler_params=pltpu.CompilerParams(needs_layout_passes=False),
        scratch_types=[
            pltpu.VMEM((2 * CHUNK,), jnp.int32),
            pltpu.VMEM((2 * CHUNK,), jnp.int32),
            pltpu.VMEM((SLOTW,), jnp.int32),
            pltpu.VMEM((SLOTW,), jnp.int32),
            pltpu.VMEM((SLOTW,), jnp.int32),
            pltpu.VMEM((SLOTW,), jnp.int32),
            pltpu.SemaphoreType.DMA,
            pltpu.SemaphoreType.DMA,
        ],
    )
    return k(src, dst)


def _sc_agg_body(h_hbm, packed_hbm, out_hbm,
                 slotb, rows, idxring, acc, semg, sems):
    c = lax.axis_index("c")
    s = lax.axis_index("s")
    wid = c * NS + s
    d = rows.shape[1]
    zero = jnp.zeros((LANES,), jnp.float32)

    for r in range(NROUND):
        w = r * (NC * NS) + wid

        @pl.when(w < NWIN)
        def _():
            base = w * WROWS

            def zbody(i, carry):
                for j in range(d // LANES):
                    acc[i, pl.ds(j * LANES, LANES)] = zero
                return carry

            lax.fori_loop(0, ACC_ROWS, zbody, 0)

            def stage(k):
                return pltpu.make_async_copy(
                    packed_hbm.at[pl.ds((w * NCHK + k) * SLOTW, SLOTW)],
                    slotb.at[pl.ds((k % 2) * SLOTW, SLOTW)], sems)

            stage(0).start()

            def slot_body(k, carry):
                soff = (k % 2) * SLOTW
                stage(k).wait()

                @pl.when(k + 1 < NCHK)
                def _():
                    stage(k + 1).start()

                nb = slotb[pl.ds(soff, LANES)][0]

                def gcopy(b):
                    return pltpu.make_async_copy(
                        h_hbm.at[idxring.at[pl.ds((b % PIPE) * GB, GB)]],
                        rows.at[pl.ds((b % PIPE) * GB, GB)], semg)

                def prep(b):
                    for g in range(GB // LANES):
                        pv = slotb[pl.ds(soff + LANES + b * GB + g * LANES,
                                         LANES)]
                        idxring[pl.ds((b % PIPE) * GB + g * LANES,
                                      LANES)] = pv & 0xFFFF
                    gcopy(b).start()

                def prebody(b, carry):
                    prep(b)
                    return carry

                lax.fori_loop(0, jnp.minimum(nb, PIPE), prebody, 0)

                def gbody(b, carry):
                    gcopy(b).wait()
                    off = (b % PIPE) * GB
                    for g in range(GB // LANES):
                        pv = slotb[pl.ds(soff + LANES + b * GB + g * LANES,
                                         LANES)]
                        ldstv = lax.shift_right_logical(pv, 16)
                        for kk in range(LANES):
                            row = ldstv[kk]
                            for j in range(d // LANES):
                                plsc.addupdate(
                                    acc.at[row, pl.ds(j * LANES, LANES)],
                                    rows[off + g * LANES + kk,
                                         pl.ds(j * LANES, LANES)])

                    @pl.when(b + PIPE < nb)
                    def _():
                        prep(b + PIPE)

                    return carry

                lax.fori_loop(0, nb, gbody, 0)
                return carry

            lax.fori_loop(0, NCHK, slot_body, 0)

            pltpu.sync_copy(acc.at[pl.ds(0, WROWS)],
                            out_hbm.at[pl.ds(base, WROWS)])


def _aggregate(h, packed):
    d = h.shape[1]
    mesh = plsc.VectorSubcoreMesh(core_axis_name="c", subcore_axis_name="s",
                                  num_cores=NC, num_subcores=NS)
    k = pl.kernel(
        _sc_agg_body,
        out_type=jax.ShapeDtypeStruct((NOUT, d), jnp.float32),
        mesh=mesh,
        compiler_params=pltpu.CompilerParams(needs_layout_passes=False),
        scratch_types=[
            pltpu.VMEM((2 * SLOTW,), jnp.int32),
            pltpu.VMEM((PIPE * GB, d), jnp.float32),
            pltpu.VMEM((PIPE * GB,), jnp.int32),
            pltpu.VMEM((ACC_ROWS, d), jnp.float32),
            pltpu.SemaphoreType.DMA,
            pltpu.SemaphoreType.DMA,
        ],
    )
    return k(h, packed)[:N]


def _mlp_body(h_ref, agg_ref, w1_ref, b1_ref, w2_ref, b2_ref, o_ref):
    z = h_ref[...] + agg_ref[...]
    t = jnp.dot(z, w1_ref[...], preferred_element_type=jnp.float32)
    t = jnp.maximum(t + b1_ref[...], 0.0)
    o = jnp.dot(t, w2_ref[...], preferred_element_type=jnp.float32)
    o_ref[...] = jnp.maximum(o + b2_ref[...], 0.0)


def _mlp(h, agg, w1, b1, w2, b2):
    din = h.shape[1]
    grid = N // BM
    return pl.pallas_call(
        _mlp_body,
        grid=(grid,),
        in_specs=[
            pl.BlockSpec((BM, din), lambda i: (i, 0)),
            pl.BlockSpec((BM, din), lambda i: (i, 0)),
            pl.BlockSpec((din, DIM), lambda i: (0, 0)),
            pl.BlockSpec((1, DIM), lambda i: (0, 0)),
            pl.BlockSpec((DIM, DIM), lambda i: (0, 0)),
            pl.BlockSpec((1, DIM), lambda i: (0, 0)),
        ],
        out_specs=pl.BlockSpec((BM, DIM), lambda i: (i, 0)),
        out_shape=jax.ShapeDtypeStruct((N, DIM), jnp.float32),
    )(h, agg, w1, b1.reshape(1, DIM), w2, b2.reshape(1, DIM))


def _mlp_pool_body(h_ref, agg_ref, w1_ref, b1_ref, w2_ref, b2_ref,
                   batch_ref, o_ref):
    z = h_ref[...] + agg_ref[...]
    t = jnp.dot(z, w1_ref[...], preferred_element_type=jnp.float32)
    t = jnp.maximum(t + b1_ref[...], 0.0)
    o = jnp.dot(t, w2_ref[...], preferred_element_type=jnp.float32)
    o = jnp.maximum(o + b2_ref[...], 0.0)
    bids = batch_ref[0]
    onehot = (jax.lax.broadcasted_iota(jnp.int32, (G, BM), 0) == bids).astype(
        jnp.float32)
    contrib = jnp.dot(onehot, o, preferred_element_type=jnp.float32)

    @pl.when(pl.program_id(0) == 0)
    def _():
        o_ref[...] = contrib

    @pl.when(pl.program_id(0) != 0)
    def _():
        o_ref[...] += contrib


def _mlp_pool(h, agg, w1, b1, w2, b2, batch3):
    din = h.shape[1]
    grid = N // BM
    return pl.pallas_call(
        _mlp_pool_body,
        grid=(grid,),
        in_specs=[
            pl.BlockSpec((BM, din), lambda i: (i, 0)),
            pl.BlockSpec((BM, din), lambda i: (i, 0)),
            pl.BlockSpec((din, DIM), lambda i: (0, 0)),
            pl.BlockSpec((1, DIM), lambda i: (0, 0)),
            pl.BlockSpec((DIM, DIM), lambda i: (0, 0)),
            pl.BlockSpec((1, DIM), lambda i: (0, 0)),
            pl.BlockSpec((1, 1, BM), lambda i: (i, 0, 0)),
        ],
        out_specs=pl.BlockSpec((G, DIM), lambda i: (0, 0)),
        out_shape=jax.ShapeDtypeStruct((G, DIM), jnp.float32),
    )(h, agg, w1, b1.reshape(1, DIM), w2, b2.reshape(1, DIM), batch3)


def kernel(x, edge_index, batch,
           W1_0, b1_0, W2_0, b2_0,
           W1_1, b1_1, W2_1, b2_1,
           W1_2, b1_2, W2_2, b2_2):
    src = edge_index[0].astype(jnp.int32)
    dst = edge_index[1].astype(jnp.int32)
    batch3 = batch.astype(jnp.int32).reshape(N // BM, 1, BM)

    packed = _partition(src, dst)
    h = x
    agg = _aggregate(h, packed)
    h = _mlp(h, agg, W1_0, b1_0, W2_0, b2_0)
    agg = _aggregate(h, packed)
    h = _mlp(h, agg, W1_1, b1_1, W2_1, b2_1)
    agg = _aggregate(h, packed)
    return _mlp_pool(h, agg, W1_2, b1_2, W2_2, b2_2, batch3)


# speedup vs baseline: 1.7637x; 1.0010x over previous
import jax
import jax.numpy as jnp
from jax import lax
from jax.experimental import pallas as pl
from jax.experimental.pallas import tpu as pltpu
from jax.experimental.pallas import tpu_sc as plsc

N = 10000
E = 160000
F_IN = 256
DIM = 512
G = 64

BM = 2000

NC = 2
NS = 16
LANES = 16

WROWS = 160
NWIN = 63
NROUND = 2
ACC_ROWS = 161
NOUT = NWIN * WROWS
CHUNK = 4000
NCHK = E // CHUNK
GRP = CHUNK // LANES
GB = 16
PIPE = 4
SLOTW = CHUNK + 3 * LANES
DUMMY = WROWS << 16


def _sc_part_body(src_hbm, dst_hbm, packed_hbm,
                  srcb, dstb, cpk0a, cpk0b, cpk1a, cpk1b, seme, semw):
    c = lax.axis_index("c")
    s = lax.axis_index("s")
    wid = c * NS + s
    base0 = wid * WROWS
    base1 = (NC * NS + wid) * WROWS
    lanesv = lax.iota(jnp.int32, LANES)
    hdr0 = jnp.full((LANES,), LANES, jnp.int32)

    def echunk(k, par):
        off = par * CHUNK
        return (pltpu.make_async_copy(
                    dst_hbm.at[pl.ds(k * CHUNK, CHUNK)],
                    dstb.at[pl.ds(off, CHUNK)], seme),
                pltpu.make_async_copy(
                    src_hbm.at[pl.ds(k * CHUNK, CHUNK)],
                    srcb.at[pl.ds(off, CHUNK)], seme))

    for cp in echunk(0, 0):
        cp.start()

    def pair_body(kk, carry):
        for par, cpk0, cpk1 in ((0, cpk0a, cpk1a), (1, cpk0b, cpk1b)):
            k = 2 * kk + par
            eoff = par * CHUNK
            for cp in echunk(k, par):
                cp.wait()

            @pl.when(k + 1 < NCHK)
            def _():
                for cp in echunk(k + 1, 1 - par):
                    cp.start()

            @pl.when(kk > 0)
            def _():
                pltpu.make_async_copy(
                    packed_hbm.at[pl.ds(0, SLOTW)], cpk0, semw).wait()
                pltpu.make_async_copy(
                    packed_hbm.at[pl.ds(0, SLOTW)], cpk1, semw).wait()

            def fbody(i, nvs):
                nv0, nv1 = nvs
                dv = dstb[pl.ds(eoff + i * LANES, LANES)]
                sv = srcb[pl.ds(eoff + i * LANES, LANES)]
                lo0 = dv - base0
                m0 = plsc.bitcast(lo0, jnp.uint32) < WROWS
                pos0 = nv0 + plsc.cumsum(jnp.where(m0, 1, 0)) - 1
                p0 = lax.shift_left(lo0, 16) | sv
                plsc.store_scatter(cpk0, [pos0], p0, mask=m0)
                lo1 = dv - base1
                m1 = plsc.bitcast(lo1, jnp.uint32) < WROWS
                pos1 = nv1 + plsc.cumsum(jnp.where(m1, 1, 0)) - 1
                p1 = lax.shift_left(lo1, 16) | sv
                plsc.store_scatter(cpk1, [pos1], p1, mask=m1)
                return (nv0 + plsc.all_reduce_population_count(m0),
                        nv1 + plsc.all_reduce_population_count(m1))

            nv0, nv1 = lax.fori_loop(0, GRP, fbody, (hdr0, hdr0))

            for cpk, nv, w in ((cpk0, nv0, wid), (cpk1, nv1, NC * NS + wid)):
                n = nv[0]
                g0 = n // LANES
                keep = lanesv < (n - g0 * LANES)
                vd = cpk[pl.ds(g0 * LANES, LANES)]
                cpk[pl.ds(g0 * LANES, LANES)] = jnp.where(keep, vd, DUMMY)
                cpk[pl.ds((g0 + 1) * LANES, LANES)] = jnp.full(
                    (LANES,), DUMMY, jnp.int32)
                cpk[pl.ds(0, LANES)] = (nv - LANES + GB - 1) // GB
                pltpu.make_async_copy(
                    cpk, packed_hbm.at[pl.ds((w * NCHK + k) * SLOTW, SLOTW)],
                    semw).start()
        return carry

    lax.fori_loop(0, NCHK // 2, pair_body, 0)

    for cpk in (cpk0a, cpk1a, cpk0b, cpk1b):
        pltpu.make_async_copy(
            packed_hbm.at[pl.ds(0, SLOTW)], cpk, semw).wait()


def _partition(src, dst):
    mesh = plsc.VectorSubcoreMesh(core_axis_name="c", subcore_axis_name="s",
                                  num_cores=NC, num_subcores=NS)
    k = pl.kernel(
        _sc_part_body,
        out_type=jax.ShapeDtypeStruct((2 * NC * NS * NCHK * SLOTW,), jnp.int32),
        mesh=mesh,
        compiler_params=pltpu.CompilerParams(needs_layout_passes=False),
        scratch_types=[
            pltpu.VMEM((2 * CHUNK,), jnp.int32),
            pltpu.VMEM((2 * CHUNK,), jnp.int32),
            pltpu.VMEM((SLOTW,), jnp.int32),
            pltpu.VMEM((SLOTW,), jnp.int32),
            pltpu.VMEM((SLOTW,), jnp.int32),
            pltpu.VMEM((SLOTW,), jnp.int32),
            pltpu.SemaphoreType.DMA,
            pltpu.SemaphoreType.DMA,
        ],
    )
    return k(src, dst)


def _sc_agg_body(h_hbm, packed_hbm, out_hbm,
                 slotb, rows, idxring, acc, semg, sems):
    c = lax.axis_index("c")
    s = lax.axis_index("s")
    wid = c * NS + s
    d = rows.shape[1]
    zero = jnp.zeros((LANES,), jnp.float32)

    for r in range(NROUND):
        w = r * (NC * NS) + wid

        @pl.when(w < NWIN)
        def _():
            base = w * WROWS

            def zbody(i, carry):
                for j in range(d // LANES):
                    acc[i, pl.ds(j * LANES, LANES)] = zero
                return carry

            lax.fori_loop(0, ACC_ROWS, zbody, 0)

            def stage(k):
                return pltpu.make_async_copy(
                    packed_hbm.at[pl.ds((w * NCHK + k) * SLOTW, SLOTW)],
                    slotb.at[pl.ds((k % 2) * SLOTW, SLOTW)], sems)

            stage(0).start()

            def slot_body(k, carry):
                soff = (k % 2) * SLOTW
                stage(k).wait()

                @pl.when(k + 1 < NCHK)
                def _():
                    stage(k + 1).start()

                nb = slotb[pl.ds(soff, LANES)][0]

                def gcopy(b):
                    return pltpu.make_async_copy(
                        h_hbm.at[idxring.at[pl.ds((b % PIPE) * GB, GB)]],
                        rows.at[pl.ds((b % PIPE) * GB, GB)], semg)

                def prep(b):
                    for g in range(GB // LANES):
                        pv = slotb[pl.ds(soff + LANES + b * GB + g * LANES,
                                         LANES)]
                        idxring[pl.ds((b % PIPE) * GB + g * LANES,
                                      LANES)] = pv & 0xFFFF
                    gcopy(b).start()

                def prebody(b, carry):
                    prep(b)
                    return carry

                lax.fori_loop(0, jnp.minimum(nb, PIPE), prebody, 0)

                def gbody(b, carry):
                    gcopy(b).wait()
                    off = (b % PIPE) * GB
                    for g in range(GB // LANES):
                        pv = slotb[pl.ds(soff + LANES + b * GB + g * LANES,
                                         LANES)]
                        ldstv = lax.shift_right_logical(pv, 16)
                        for kk in range(LANES):
                            row = ldstv[kk]
                            for j in range(d // LANES):
                                plsc.addupdate(
                                    acc.at[row, pl.ds(j * LANES, LANES)],
                                    rows[off + g * LANES + kk,
                                         pl.ds(j * LANES, LANES)])

                    @pl.when(b + PIPE < nb)
                    def _():
                        prep(b + PIPE)

                    return carry

                lax.fori_loop(0, nb, gbody, 0)
                return carry

            lax.fori_loop(0, NCHK, slot_body, 0)

            pltpu.sync_copy(acc.at[pl.ds(0, WROWS)],
                            out_hbm.at[pl.ds(base, WROWS)])


def _aggregate(h, packed):
    d = h.shape[1]
    mesh = plsc.VectorSubcoreMesh(core_axis_name="c", subcore_axis_name="s",
                                  num_cores=NC, num_subcores=NS)
    k = pl.kernel(
        _sc_agg_body,
        out_type=jax.ShapeDtypeStruct((NOUT, d), jnp.float32),
        mesh=mesh,
        compiler_params=pltpu.CompilerParams(needs_layout_passes=False),
        scratch_types=[
            pltpu.VMEM((2 * SLOTW,), jnp.int32),
            pltpu.VMEM((PIPE * GB, d), jnp.float32),
            pltpu.VMEM((PIPE * GB,), jnp.int32),
            pltpu.VMEM((ACC_ROWS, d), jnp.float32),
            pltpu.SemaphoreType.DMA,
            pltpu.SemaphoreType.DMA,
        ],
    )
    return k(h, packed)[:N]


def _mlp_body(h_ref, agg_ref, w1_ref, b1_ref, w2_ref, b2_ref, o_ref):
    z = h_ref[...] + agg_ref[...]
    t = jnp.dot(z, w1_ref[...], preferred_element_type=jnp.float32)
    t = jnp.maximum(t + b1_ref[...], 0.0)
    o = jnp.dot(t, w2_ref[...], preferred_element_type=jnp.float32)
    o_ref[...] = jnp.maximum(o + b2_ref[...], 0.0)


def _mlp(h, agg, w1, b1, w2, b2):
    din = h.shape[1]
    grid = N // BM
    return pl.pallas_call(
        _mlp_body,
        grid=(grid,),
        in_specs=[
            pl.BlockSpec((BM, din), lambda i: (i, 0)),
            pl.BlockSpec((BM, din), lambda i: (i, 0)),
            pl.BlockSpec((din, DIM), lambda i: (0, 0)),
            pl.BlockSpec((1, DIM), lambda i: (0, 0)),
            pl.BlockSpec((DIM, DIM), lambda i: (0, 0)),
            pl.BlockSpec((1, DIM), lambda i: (0, 0)),
        ],
        out_specs=pl.BlockSpec((BM, DIM), lambda i: (i, 0)),
        out_shape=jax.ShapeDtypeStruct((N, DIM), jnp.float32),
    )(h, agg, w1, b1.reshape(1, DIM), w2, b2.reshape(1, DIM))


def _mlp_pool_body(h_ref, agg_ref, w1_ref, b1_ref, w2_ref, b2_ref,
                   batch_ref, o_ref):
    z = h_ref[...] + agg_ref[...]
    t = jnp.dot(z, w1_ref[...], preferred_element_type=jnp.float32)
    t = jnp.maximum(t + b1_ref[...], 0.0)
    o = jnp.dot(t, w2_ref[...], preferred_element_type=jnp.float32)
    o = jnp.maximum(o + b2_ref[...], 0.0)
    bids = batch_ref[0]
    onehot = (jax.lax.broadcasted_iota(jnp.int32, (G, BM), 0) == bids).astype(
        jnp.float32)
    contrib = jnp.dot(onehot, o, preferred_element_type=jnp.float32)

    @pl.when(pl.program_id(0) == 0)
    def _():
        o_ref[...] = contrib

    @pl.when(pl.program_id(0) != 0)
    def _():
        o_ref[...] += contrib


def _mlp_pool(h, agg, w1, b1, w2, b2, batch3):
    din = h.shape[1]
    grid = N // BM
    return pl.pallas_call(
        _mlp_pool_body,
        grid=(grid,),
        in_specs=[
            pl.BlockSpec((BM, din), lambda i: (i, 0)),
            pl.BlockSpec((BM, din), lambda i: (i, 0)),
            pl.BlockSpec((din, DIM), lambda i: (0, 0)),
            pl.BlockSpec((1, DIM), lambda i: (0, 0)),
            pl.BlockSpec((DIM, DIM), lambda i: (0, 0)),
            pl.BlockSpec((1, DIM), lambda i: (0, 0)),
            pl.BlockSpec((1, 1, BM), lambda i: (i, 0, 0)),
        ],
        out_specs=pl.BlockSpec((G, DIM), lambda i: (0, 0)),
        out_shape=jax.ShapeDtypeStruct((G, DIM), jnp.float32),
    )(h, agg, w1, b1.reshape(1, DIM), w2, b2.reshape(1, DIM), batch3)


def kernel(x, edge_index, batch,
           W1_0, b1_0, W2_0, b2_0,
           W1_1, b1_1, W2_1, b2_1,
           W1_2, b1_2, W2_2, b2_2):
    src = edge_index[0].astype(jnp.int32)
    dst = edge_index[1].astype(jnp.int32)
    batch3 = batch.astype(jnp.int32).reshape(N // BM, 1, BM)

    packed = _partition(src, dst)
    h = x
    agg = _aggregate(h, packed)
    h = _mlp(h, agg, W1_0, b1_0, W2_0, b2_0)
    agg = _aggregate(h, packed)
    h = _mlp(h, agg, W1_1, b1_1, W2_1, b2_1)
    agg = _aggregate(h, packed)
    return _mlp_pool(h, agg, W1_2, b1_2, W2_2, b2_2, batch3)
